# baseline (speedup 1.0000x reference)
import functools

import jax
import jax.numpy as jnp
from jax import lax
from jax.experimental import pallas as pl
from jax.experimental.pallas import tpu as pltpu
from jax.experimental.pallas import tpu_sc as plsc

N = 10000
NP_ = 10240
E = 320000
D = 128
DH = 64
DEC = 32

NC = 2
NS = 16
CH = 80
NCHUNK = E // CH
CPT1 = NCHUNK // NS
CPT3 = NCHUNK // (NC * NS)
RPT = NP_ // NS

_MESH = plsc.VectorSubcoreMesh(
    core_axis_name="c", subcore_axis_name="s", num_cores=NC, num_subcores=NS)
_SC_PARAMS = pltpu.CompilerParams(
    use_tc_tiling_on_sc=False, needs_layout_passes=False)

_f32 = jnp.float32
_i32 = jnp.int32


@functools.partial(
    pl.kernel,
    out_type=(
        jax.ShapeDtypeStruct((NC, 2, NP_, DH), _f32),
        jax.ShapeDtypeStruct((NC, NP_, 8), _f32),
    ),
    mesh=_MESH,
    compiler_params=_SC_PARAMS,
    scratch_types=(
        pltpu.VMEM((CPT1, CH), _i32),
        pltpu.VMEM((CPT1, CH), _i32),
        pltpu.VMEM((CH, DH), _f32),
        pltpu.VMEM((CH, DH), _f32),
        pltpu.VMEM((CH, 8), _f32),
        pltpu.VMEM_SHARED((NP_, DH), _f32),
        pltpu.VMEM_SHARED((NP_, 8), _f32),
        pltpu.SemaphoreType.DMA,
        pltpu.SemaphoreType.DMA,
        pltpu.SemaphoreType.DMA,
        pltpu.SemaphoreType.DMA,
        pltpu.SemaphoreType.DMA,
        pltpu.SemaphoreType.DMA,
    ),
)
def _aggregate(xlo, xhi, srcs, dsts, z2d, z8, ones8,
               sums, cnts,
               sidx_v, didx_v, rows_v0, rows_v1, ones_v, acc_sh, cnt_sh,
               sem_g0, sem_g1, sem_s0, sem_s1, sem_c0, sem_c1):
    cid = lax.axis_index("c")
    sid = lax.axis_index("s")
    n0 = sid * RPT
    pltpu.sync_copy(srcs.at[cid].at[sid], sidx_v)
    pltpu.sync_copy(dsts.at[cid].at[sid], didx_v)
    pltpu.sync_copy(ones8, ones_v)

    for half, xh in enumerate((xlo, xhi)):
        pltpu.sync_copy(z2d, acc_sh.at[pl.ds(n0, RPT)])
        if half == 0:
            pltpu.sync_copy(z8, cnt_sh.at[pl.ds(n0, RPT)])
        plsc.subcore_barrier()

        @pl.loop(0, CPT1 // 2)
        def _(s):
            c0 = 2 * s
            c1 = c0 + 1

            @pl.when(s > 0)
            def _():
                pltpu.make_async_copy(
                    rows_v0, acc_sh.at[didx_v.at[c0]], sem_s0).wait()
                if half == 0:
                    pltpu.make_async_copy(
                        ones_v, cnt_sh.at[didx_v.at[c0]], sem_c0).wait()

            g0 = pltpu.async_copy(xh.at[sidx_v.at[c0]], rows_v0, sem_g0)

            @pl.when(s > 0)
            def _():
                pltpu.make_async_copy(
                    rows_v1, acc_sh.at[didx_v.at[c1]], sem_s1).wait()
                if half == 0:
                    pltpu.make_async_copy(
                        ones_v, cnt_sh.at[didx_v.at[c1]], sem_c1).wait()

            g1 = pltpu.async_copy(xh.at[sidx_v.at[c1]], rows_v1, sem_g1)

            g0.wait()
            pltpu.async_copy(rows_v0, acc_sh.at[didx_v.at[c0]], sem_s0,
                             add=True)
            if half == 0:
                pltpu.async_copy(ones_v, cnt_sh.at[didx_v.at[c0]], sem_c0,
                                 add=True)
            g1.wait()
            pltpu.async_copy(rows_v1, acc_sh.at[didx_v.at[c1]], sem_s1,
                             add=True)
            if half == 0:
                pltpu.async_copy(ones_v, cnt_sh.at[didx_v.at[c1]], sem_c1,
                                 add=True)

        pltpu.make_async_copy(
            rows_v0, acc_sh.at[didx_v.at[CPT1 - 2]], sem_s0).wait()
        pltpu.make_async_copy(
            rows_v1, acc_sh.at[didx_v.at[CPT1 - 1]], sem_s1).wait()
        if half == 0:
            pltpu.make_async_copy(
                ones_v, cnt_sh.at[didx_v.at[CPT1 - 2]], sem_c0).wait()
            pltpu.make_async_copy(
                ones_v, cnt_sh.at[didx_v.at[CPT1 - 1]], sem_c1).wait()
        plsc.subcore_barrier()
        pltpu.sync_copy(acc_sh.at[pl.ds(n0, RPT)],
                        sums.at[cid].at[half].at[pl.ds(n0, RPT)])
        if half == 0:
            pltpu.sync_copy(cnt_sh.at[pl.ds(n0, RPT)],
                            cnts.at[cid].at[pl.ds(n0, RPT)])


_R2 = 1000


def _encode_body(sflo, sfhi, cnt_f, sblo, sbhi, cnt_b, x,
                 wlf, wrf, wlb, wrb, blf, brf, blb, brb, w1a, w1b, b1r,
                 a_out, b_out):
    sf = jnp.concatenate([sflo[...], sfhi[...]], axis=1)
    sb = jnp.concatenate([sblo[...], sbhi[...]], axis=1)
    mf = sf / jnp.maximum(cnt_f[...][:, 0:1], 1.0)
    mb = sb / jnp.maximum(cnt_b[...][:, 0:1], 1.0)
    xb = x[...]
    x1 = (jnp.dot(mf, wlf[...], preferred_element_type=_f32)
          + jnp.dot(xb, wrf[...], preferred_element_type=_f32)
          + blf[...] + brf[...])
    x2 = (jnp.dot(mb, wlb[...], preferred_element_type=_f32)
          + jnp.dot(xb, wrb[...], preferred_element_type=_f32)
          + blb[...] + brb[...])
    z = jnp.maximum((x1 + x2) * 0.5, 0.0)
    halfb1 = 0.5 * b1r[...]
    a_out[...] = jnp.dot(z, w1a[...], preferred_element_type=_f32) + halfb1
    b_out[...] = jnp.dot(z, w1b[...], preferred_element_type=_f32) + halfb1


def _encode(sflo, sfhi, cnt_f, sblo, sbhi, cnt_b, x, wlf, wrf, wlb, wrb,
            blf, brf, blb, brb, w1a, w1b, b1r):
    row_spec = pl.BlockSpec((_R2, D), lambda i: (i, 0))
    half_spec = pl.BlockSpec((_R2, DH), lambda i: (i, 0))
    cnt_spec = pl.BlockSpec((_R2, 8), lambda i: (i, 0))
    w_spec = pl.BlockSpec((D, D), lambda i: (0, 0))
    wh_spec = pl.BlockSpec((D, DEC), lambda i: (0, 0))
    b_spec = pl.BlockSpec((1, D), lambda i: (0, 0))
    b1_spec = pl.BlockSpec((1, DEC), lambda i: (0, 0))
    out_spec = pl.BlockSpec((_R2, DEC), lambda i: (i, 0))
    return pl.pallas_call(
        _encode_body,
        grid=(N // _R2,),
        in_specs=[half_spec, half_spec, cnt_spec,
                  half_spec, half_spec, cnt_spec, row_spec,
                  w_spec, w_spec, w_spec, w_spec,
                  b_spec, b_spec, b_spec, b_spec,
                  wh_spec, wh_spec, b1_spec],
        out_specs=[out_spec, out_spec],
        out_shape=[jax.ShapeDtypeStruct((N, DEC), _f32),
                   jax.ShapeDtypeStruct((N, DEC), _f32)],
    )(sflo, sfhi, cnt_f, sblo, sbhi, cnt_b, x, wlf, wrf, wlb, wrb,
      blf, brf, blb, brb, w1a, w1b, b1r)


@functools.partial(
    pl.kernel,
    out_type=(
        jax.ShapeDtypeStruct((E, DEC), _f32),
        jax.ShapeDtypeStruct((E, DEC), _f32),
    ),
    mesh=_MESH,
    compiler_params=_SC_PARAMS,
    scratch_types=(
        pltpu.VMEM((CPT3, CH), _i32),
        pltpu.VMEM((CPT3, CH), _i32),
        pltpu.VMEM((CH, DEC), _f32),
        pltpu.VMEM((CH, DEC), _f32),
        pltpu.VMEM((CH, DEC), _f32),
        pltpu.VMEM((CH, DEC), _f32),
        pltpu.SemaphoreType.DMA,
        pltpu.SemaphoreType.DMA,
        pltpu.SemaphoreType.DMA,
        pltpu.SemaphoreType.DMA,
        pltpu.SemaphoreType.DMA,
        pltpu.SemaphoreType.DMA,
    ),
)
def _edge_gather(a_hbm, b_hbm, esrc, edst, ga, gb,
                 sidx_v, didx_v, a_v0, a_v1, b_v0, b_v1,
                 sem_a0, sem_a1, sem_b0, sem_b1, sem_s0, sem_s1):
    cid = lax.axis_index("c")
    sid = lax.axis_index("s")
    wid = cid * NS + sid
    r0 = wid * CPT3
    pltpu.sync_copy(esrc.at[wid], sidx_v)
    pltpu.sync_copy(edst.at[wid], didx_v)

    def start_gather(c, a_v, b_v, sem_a, sem_b):
        pltpu.async_copy(a_hbm.at[sidx_v.at[c]], a_v, sem_a)
        pltpu.async_copy(b_hbm.at[didx_v.at[c]], b_v, sem_b)

    def wait_gather(c, a_v, b_v, sem_a, sem_b):
        pltpu.make_async_copy(a_hbm.at[sidx_v.at[c]], a_v, sem_a).wait()
        pltpu.make_async_copy(b_hbm.at[didx_v.at[c]], b_v, sem_b).wait()

    def store(c, a_v, b_v, sem_s):
        e0 = (r0 + c) * CH
        pltpu.async_copy(a_v, ga.at[pl.ds(e0, CH)], sem_s)
        pltpu.async_copy(b_v, gb.at[pl.ds(e0, CH)], sem_s)
        pltpu.make_async_copy(a_v, ga.at[pl.ds(e0, CH)], sem_s).wait()
        pltpu.make_async_copy(b_v, gb.at[pl.ds(e0, CH)], sem_s).wait()

    start_gather(0, a_v0, b_v0, sem_a0, sem_b0)
    start_gather(1, a_v1, b_v1, sem_a1, sem_b1)

    NSTEP = CPT3 // 2

    @pl.loop(0, NSTEP)
    def _(s):
        c0 = 2 * s
        c1 = c0 + 1
        wait_gather(c0, a_v0, b_v0, sem_a0, sem_b0)
        store(c0, a_v0, b_v0, sem_s0)
        pltpu.async_copy(a_hbm.at[sidx_v.at[c0 + 2]], a_v0, sem_a0)
        pltpu.async_copy(b_hbm.at[didx_v.at[c0 + 2]], b_v0, sem_b0)

        wait_gather(c1, a_v1, b_v1, sem_a1, sem_b1)
        store(c1, a_v1, b_v1, sem_s1)

        @pl.when(s + 1 < NSTEP)
        def _():
            pltpu.async_copy(a_hbm.at[sidx_v.at[c1 + 2]], a_v1, sem_a1)
            pltpu.async_copy(b_hbm.at[didx_v.at[c1 + 2]], b_v1, sem_b1)

    last = CPT3 - 1
    wait_gather(last, a_v0, b_v0, sem_a0, sem_b0)
    store(last, a_v0, b_v0, sem_s0)


_RD = 6400


def _head_body(ga, gb, w2r, b2r, o):
    h = jnp.maximum(ga[...] + gb[...], 0.0)
    t = jnp.dot(h, w2r[...], preferred_element_type=_f32) + b2r[...]
    o[...] = 1.0 / (1.0 + jnp.exp(-t))


def _head(ga, gb, w2, b2):
    e_spec = pl.BlockSpec((_RD, DEC), lambda i: (i, 0))
    return pl.pallas_call(
        _head_body,
        grid=(E // _RD,),
        in_specs=[e_spec, e_spec,
                  pl.BlockSpec((DEC, 1), lambda i: (0, 0)),
                  pl.BlockSpec((1, 1), lambda i: (0, 0))],
        out_specs=pl.BlockSpec((_RD, 1), lambda i: (i, 0)),
        out_shape=jax.ShapeDtypeStruct((E, 1), _f32),
    )(ga, gb, w2, b2)


def kernel(x, adj, batch, edge_index,
           Wl_f, bl_f, Wr_f, br_f,
           Wl_b, bl_b, Wr_b, br_b,
           W1, b1, W2, b2):
    srcs = jnp.stack([adj[0].astype(_i32).reshape(NS, CPT1, CH),
                      batch[0].astype(_i32).reshape(NS, CPT1, CH)])
    dsts = jnp.stack([adj[1].astype(_i32).reshape(NS, CPT1, CH),
                      batch[1].astype(_i32).reshape(NS, CPT1, CH)])
    esrc = edge_index[0].astype(_i32).reshape(NC * NS, CPT3, CH)
    edst = edge_index[1].astype(_i32).reshape(NC * NS, CPT3, CH)

    xlo = x[:, :DH]
    xhi = x[:, DH:]

    z2d = jnp.zeros((RPT, DH), _f32)
    z8 = jnp.zeros((RPT, 8), _f32)
    ones8 = jnp.ones((CH, 8), _f32)

    sums, cnts = _aggregate(xlo, xhi, srcs, dsts, z2d, z8, ones8)

    a_p, b_p = _encode(
        sums[0, 0], sums[0, 1], cnts[0], sums[1, 0], sums[1, 1], cnts[1], x,
        Wl_f, Wr_f, Wl_b, Wr_b,
        bl_f.reshape(1, D), br_f.reshape(1, D),
        bl_b.reshape(1, D), br_b.reshape(1, D),
        W1[:D], W1[D:], b1.reshape(1, DEC))

    ga, gb = _edge_gather(a_p, b_p, esrc, edst)
    return _head(ga, gb, W2, b2.reshape(1, 1))[:, 0]


# speedup vs baseline: 1.6890x; 1.6890x over previous
import functools

import jax
import jax.numpy as jnp
from jax import lax
from jax.experimental import pallas as pl
from jax.experimental.pallas import tpu as pltpu
from jax.experimental.pallas import tpu_sc as plsc

N = 10000
NP_ = 10240
E = 320000
D = 128
DH = 64
DEC = 32

NC = 2
NS = 16
CH = 80
NCHUNK = E // CH
CPT1 = NCHUNK // NS
CPT3 = NCHUNK // (NC * NS)
RPT = NP_ // NS

_MESH = plsc.VectorSubcoreMesh(
    core_axis_name="c", subcore_axis_name="s", num_cores=NC, num_subcores=NS)
_SC_PARAMS = pltpu.CompilerParams(
    use_tc_tiling_on_sc=False, needs_layout_passes=False)

_f32 = jnp.float32
_i32 = jnp.int32


@functools.partial(
    pl.kernel,
    out_type=(
        jax.ShapeDtypeStruct((NC, 2, NP_, DH), _f32),
        jax.ShapeDtypeStruct((NC, NP_, 8), _f32),
    ),
    mesh=_MESH,
    compiler_params=_SC_PARAMS,
    scratch_types=(
        pltpu.VMEM((CPT1, CH), _i32),
        pltpu.VMEM((CPT1, CH), _i32),
        pltpu.VMEM((CH, DH), _f32),
        pltpu.VMEM((CH, DH), _f32),
        pltpu.VMEM((CH, 8), _f32),
        pltpu.VMEM_SHARED((NP_, DH), _f32),
        pltpu.VMEM_SHARED((NP_, 8), _f32),
        pltpu.SemaphoreType.DMA,
        pltpu.SemaphoreType.DMA,
        pltpu.SemaphoreType.DMA,
        pltpu.SemaphoreType.DMA,
        pltpu.SemaphoreType.DMA,
        pltpu.SemaphoreType.DMA,
    ),
)
def _aggregate(xlo, xhi, srcs, dsts, z2d, z8, ones8,
               sums, cnts,
               sidx_v, didx_v, rows_v0, rows_v1, ones_v, acc_sh, cnt_sh,
               sem_g0, sem_g1, sem_s0, sem_s1, sem_c0, sem_c1):
    cid = lax.axis_index("c")
    sid = lax.axis_index("s")
    n0 = sid * RPT
    pltpu.sync_copy(srcs.at[cid].at[sid], sidx_v)
    pltpu.sync_copy(dsts.at[cid].at[sid], didx_v)
    pltpu.sync_copy(ones8, ones_v)

    for half, xh in enumerate((xlo, xhi)):
        pltpu.sync_copy(z2d, acc_sh.at[pl.ds(n0, RPT)])
        if half == 0:
            pltpu.sync_copy(z8, cnt_sh.at[pl.ds(n0, RPT)])
        plsc.subcore_barrier()

        @pl.loop(0, CPT1 // 2)
        def _(s):
            c0 = 2 * s
            c1 = c0 + 1

            @pl.when(s > 0)
            def _():
                pltpu.make_async_copy(
                    rows_v0, acc_sh.at[didx_v.at[c0]], sem_s0).wait()
                if half == 0:
                    pltpu.make_async_copy(
                        ones_v, cnt_sh.at[didx_v.at[c0]], sem_c0).wait()

            g0 = pltpu.async_copy(xh.at[sidx_v.at[c0]], rows_v0, sem_g0)

            @pl.when(s > 0)
            def _():
                pltpu.make_async_copy(
                    rows_v1, acc_sh.at[didx_v.at[c1]], sem_s1).wait()
                if half == 0:
                    pltpu.make_async_copy(
                        ones_v, cnt_sh.at[didx_v.at[c1]], sem_c1).wait()

            g1 = pltpu.async_copy(xh.at[sidx_v.at[c1]], rows_v1, sem_g1)

            g0.wait()
            pltpu.async_copy(rows_v0, acc_sh.at[didx_v.at[c0]], sem_s0,
                             add=True)
            if half == 0:
                pltpu.async_copy(ones_v, cnt_sh.at[didx_v.at[c0]], sem_c0,
                                 add=True)
            g1.wait()
            pltpu.async_copy(rows_v1, acc_sh.at[didx_v.at[c1]], sem_s1,
                             add=True)
            if half == 0:
                pltpu.async_copy(ones_v, cnt_sh.at[didx_v.at[c1]], sem_c1,
                                 add=True)

        pltpu.make_async_copy(
            rows_v0, acc_sh.at[didx_v.at[CPT1 - 2]], sem_s0).wait()
        pltpu.make_async_copy(
            rows_v1, acc_sh.at[didx_v.at[CPT1 - 1]], sem_s1).wait()
        if half == 0:
            pltpu.make_async_copy(
                ones_v, cnt_sh.at[didx_v.at[CPT1 - 2]], sem_c0).wait()
            pltpu.make_async_copy(
                ones_v, cnt_sh.at[didx_v.at[CPT1 - 1]], sem_c1).wait()
        plsc.subcore_barrier()
        pltpu.sync_copy(acc_sh.at[pl.ds(n0, RPT)],
                        sums.at[cid].at[half].at[pl.ds(n0, RPT)])
        if half == 0:
            pltpu.sync_copy(cnt_sh.at[pl.ds(n0, RPT)],
                            cnts.at[cid].at[pl.ds(n0, RPT)])


_R2 = 1000


def _encode_body(sflo, sfhi, cnt_f, sblo, sbhi, cnt_b, x,
                 wlf, wrf, wlb, wrb, blf, brf, blb, brb, w1a, w1b, b1r,
                 a_out, b_out):
    sf = jnp.concatenate([sflo[...], sfhi[...]], axis=1)
    sb = jnp.concatenate([sblo[...], sbhi[...]], axis=1)
    mf = sf / jnp.maximum(cnt_f[...][:, 0:1], 1.0)
    mb = sb / jnp.maximum(cnt_b[...][:, 0:1], 1.0)
    xb = x[...]
    x1 = (jnp.dot(mf, wlf[...], preferred_element_type=_f32)
          + jnp.dot(xb, wrf[...], preferred_element_type=_f32)
          + blf[...] + brf[...])
    x2 = (jnp.dot(mb, wlb[...], preferred_element_type=_f32)
          + jnp.dot(xb, wrb[...], preferred_element_type=_f32)
          + blb[...] + brb[...])
    z = jnp.maximum((x1 + x2) * 0.5, 0.0)
    halfb1 = 0.5 * b1r[...]
    a_out[...] = jnp.dot(z, w1a[...], preferred_element_type=_f32) + halfb1
    b_out[...] = jnp.dot(z, w1b[...], preferred_element_type=_f32) + halfb1


def _encode(sflo, sfhi, cnt_f, sblo, sbhi, cnt_b, x, wlf, wrf, wlb, wrb,
            blf, brf, blb, brb, w1a, w1b, b1r):
    row_spec = pl.BlockSpec((_R2, D), lambda i: (i, 0))
    half_spec = pl.BlockSpec((_R2, DH), lambda i: (i, 0))
    cnt_spec = pl.BlockSpec((_R2, 8), lambda i: (i, 0))
    w_spec = pl.BlockSpec((D, D), lambda i: (0, 0))
    wh_spec = pl.BlockSpec((D, DEC), lambda i: (0, 0))
    b_spec = pl.BlockSpec((1, D), lambda i: (0, 0))
    b1_spec = pl.BlockSpec((1, DEC), lambda i: (0, 0))
    out_spec = pl.BlockSpec((_R2, DEC), lambda i: (i, 0))
    return pl.pallas_call(
        _encode_body,
        grid=(N // _R2,),
        in_specs=[half_spec, half_spec, cnt_spec,
                  half_spec, half_spec, cnt_spec, row_spec,
                  w_spec, w_spec, w_spec, w_spec,
                  b_spec, b_spec, b_spec, b_spec,
                  wh_spec, wh_spec, b1_spec],
        out_specs=[out_spec, out_spec],
        out_shape=[jax.ShapeDtypeStruct((N, DEC), _f32),
                   jax.ShapeDtypeStruct((N, DEC), _f32)],
    )(sflo, sfhi, cnt_f, sblo, sbhi, cnt_b, x, wlf, wrf, wlb, wrb,
      blf, brf, blb, brb, w1a, w1b, b1r)


@functools.partial(
    pl.kernel,
    out_type=jax.ShapeDtypeStruct((E,), _f32),
    mesh=_MESH,
    compiler_params=_SC_PARAMS,
    scratch_types=(
        pltpu.VMEM((CPT3, CH), _i32),
        pltpu.VMEM((CPT3, CH), _i32),
        pltpu.VMEM((CH, DEC), _f32),
        pltpu.VMEM((CH, DEC), _f32),
        pltpu.VMEM((CH, DEC), _f32),
        pltpu.VMEM((CH, DEC), _f32),
        pltpu.VMEM((CH,), _f32),
        pltpu.VMEM((CH,), _f32),
        pltpu.VMEM((48,), _f32),
        pltpu.SemaphoreType.DMA,
        pltpu.SemaphoreType.DMA,
        pltpu.SemaphoreType.DMA,
        pltpu.SemaphoreType.DMA,
        pltpu.SemaphoreType.DMA,
        pltpu.SemaphoreType.DMA,
    ),
)
def _decode(a_hbm, b_hbm, esrc, edst, params, out,
            sidx_v, didx_v, a_v0, a_v1, b_v0, b_v1, out_v0, out_v1, p_v,
            sem_a0, sem_a1, sem_b0, sem_b1, sem_o0, sem_o1):
    cid = lax.axis_index("c")
    sid = lax.axis_index("s")
    wid = cid * NS + sid
    r0 = wid * CPT3
    pltpu.sync_copy(esrc.at[wid], sidx_v)
    pltpu.sync_copy(edst.at[wid], didx_v)
    pltpu.sync_copy(params, p_v)

    lanes = lax.iota(_i32, 16)
    w2_lo = p_v[pl.ds(0, 16)]
    w2_hi = p_v[pl.ds(16, 16)]
    tail = p_v[pl.ds(32, 16)]
    b2s = tail[0]

    NG = CH // 16

    def compute(a_v, b_v, out_v):
        rows = [lanes + (16 * g) for g in range(NG)]
        accs = [jnp.zeros((16,), _f32) for _ in range(NG)]
        for j in range(DEC):
            colj = (lanes + j) & (DEC - 1)
            w2v = plsc.load_gather(p_v, [colj])
            for g in range(NG):
                a = plsc.load_gather(a_v, [rows[g], colj])
                b = plsc.load_gather(b_v, [rows[g], colj])
                accs[g] = accs[g] + jnp.maximum(a + b, 0.0) * w2v
        for g in range(NG):
            t = accs[g] + b2s
            out_v[pl.ds(16 * g, 16)] = 1.0 / (1.0 + jnp.exp(-t))

    pltpu.async_copy(a_hbm.at[sidx_v.at[0]], a_v0, sem_a0)
    pltpu.async_copy(b_hbm.at[didx_v.at[0]], b_v0, sem_b0)
    pltpu.async_copy(a_hbm.at[sidx_v.at[1]], a_v1, sem_a1)
    pltpu.async_copy(b_hbm.at[didx_v.at[1]], b_v1, sem_b1)

    NSTEP = CPT3 // 2

    @pl.loop(0, NSTEP)
    def _(s):
        c0 = 2 * s
        c1 = c0 + 1
        pltpu.make_async_copy(a_hbm.at[sidx_v.at[c0]], a_v0, sem_a0).wait()
        pltpu.make_async_copy(b_hbm.at[didx_v.at[c0]], b_v0, sem_b0).wait()

        @pl.when(s > 0)
        def _():
            pltpu.make_async_copy(
                out_v0, out.at[pl.ds((r0 + c0 - 2) * CH, CH)], sem_o0).wait()

        compute(a_v0, b_v0, out_v0)
        pltpu.async_copy(out_v0, out.at[pl.ds((r0 + c0) * CH, CH)], sem_o0)
        pltpu.async_copy(a_hbm.at[sidx_v.at[c0 + 2]], a_v0, sem_a0)
        pltpu.async_copy(b_hbm.at[didx_v.at[c0 + 2]], b_v0, sem_b0)

        pltpu.make_async_copy(a_hbm.at[sidx_v.at[c1]], a_v1, sem_a1).wait()
        pltpu.make_async_copy(b_hbm.at[didx_v.at[c1]], b_v1, sem_b1).wait()

        @pl.when(s > 0)
        def _():
            pltpu.make_async_copy(
                out_v1, out.at[pl.ds((r0 + c1 - 2) * CH, CH)], sem_o1).wait()

        compute(a_v1, b_v1, out_v1)
        pltpu.async_copy(out_v1, out.at[pl.ds((r0 + c1) * CH, CH)], sem_o1)

        @pl.when(s + 1 < NSTEP)
        def _():
            pltpu.async_copy(a_hbm.at[sidx_v.at[c1 + 2]], a_v1, sem_a1)
            pltpu.async_copy(b_hbm.at[didx_v.at[c1 + 2]], b_v1, sem_b1)

    last = CPT3 - 1
    pltpu.make_async_copy(a_hbm.at[sidx_v.at[last]], a_v0, sem_a0).wait()
    pltpu.make_async_copy(b_hbm.at[didx_v.at[last]], b_v0, sem_b0).wait()
    pltpu.make_async_copy(
        out_v0, out.at[pl.ds((r0 + last - 2) * CH, CH)], sem_o0).wait()
    compute(a_v0, b_v0, out_v0)
    pltpu.make_async_copy(
        out_v1, out.at[pl.ds((r0 + last - 1) * CH, CH)], sem_o1).wait()
    pltpu.sync_copy(out_v0, out.at[pl.ds((r0 + last) * CH, CH)])


def kernel(x, adj, batch, edge_index,
           Wl_f, bl_f, Wr_f, br_f,
           Wl_b, bl_b, Wr_b, br_b,
           W1, b1, W2, b2):
    srcs = jnp.stack([adj[0].astype(_i32).reshape(NS, CPT1, CH),
                      batch[0].astype(_i32).reshape(NS, CPT1, CH)])
    dsts = jnp.stack([adj[1].astype(_i32).reshape(NS, CPT1, CH),
                      batch[1].astype(_i32).reshape(NS, CPT1, CH)])
    esrc = edge_index[0].astype(_i32).reshape(NC * NS, CPT3, CH)
    edst = edge_index[1].astype(_i32).reshape(NC * NS, CPT3, CH)

    xlo = x[:, :DH]
    xhi = x[:, DH:]

    z2d = jnp.zeros((RPT, DH), _f32)
    z8 = jnp.zeros((RPT, 8), _f32)
    ones8 = jnp.ones((CH, 8), _f32)

    sums, cnts = _aggregate(xlo, xhi, srcs, dsts, z2d, z8, ones8)

    a_p, b_p = _encode(
        sums[0, 0], sums[0, 1], cnts[0], sums[1, 0], sums[1, 1], cnts[1], x,
        Wl_f, Wr_f, Wl_b, Wr_b,
        bl_f.reshape(1, D), br_f.reshape(1, D),
        bl_b.reshape(1, D), br_b.reshape(1, D),
        W1[:D], W1[D:], b1.reshape(1, DEC))

    params = jnp.concatenate([W2[:, 0], b2, jnp.zeros((15,), _f32)])

    return _decode(a_p, b_p, esrc, edst, params)
